# baseline (speedup 1.0000x reference)
import jax
import jax.numpy as jnp
import numpy as np
from jax import lax
from jax.experimental import pallas as pl
from jax.experimental.pallas import tpu as pltpu
from jax.experimental.pallas import tpu_sc as plsc

B = 16384
K = 20
HID = 64
HPK = HID // 2

NC = 2
NS = 16
NW = NC * NS
BPW = B // NW
CHUNK = 64
NCHUNK = BPW // CHUNK
CROWS = CHUNK * K
NSUB = CHUNK // 16
MASK_HI = np.int32(-65536)


def _fold(acc):
    ai = plsc.bitcast(acc, jnp.int32)
    lo = plsc.bitcast(lax.shift_left(ai, 16), jnp.float32)
    hi = plsc.bitcast(jnp.bitwise_and(ai, MASK_HI), jnp.float32)
    return lo + hi


def _sc_dots_body(wemb_hbm, cemb_hbm, widx_hbm, cidx_hbm, out_hbm,
                  widx_v, cidx_v, wrows_v, crows_v, res_v, sem0, sem1, sem2):
    wid = lax.axis_index("s") * NC + lax.axis_index("c")
    lanes = lax.iota(jnp.int32, 16)

    def chunk_body(i, _):
        base = wid * BPW + i * CHUNK
        pltpu.sync_copy(widx_hbm.at[pl.ds(base, CHUNK)], widx_v)
        idx_cps = [
            pltpu.async_copy(cidx_hbm.at[k, pl.ds(base, CHUNK)],
                             cidx_v.at[pl.ds(k * CHUNK, CHUNK)], sem2)
            for k in range(K)]
        for cp in idx_cps:
            cp.wait()
        wcp = pltpu.async_copy(wemb_hbm.at[widx_v], wrows_v, sem0)
        ccps = [
            pltpu.async_copy(cemb_hbm.at[cidx_v.at[pl.ds(j * 128, 128)]],
                             crows_v.at[pl.ds(j * 128, 128)], sem1)
            for j in range(CROWS // 128)]
        wcp.wait()
        for cp in ccps:
            cp.wait()

        for sub in range(NSUB):
            brow = lanes + sub * 16
            ridx = [brow + (k * CHUNK + sub * 16) for k in range(K)]

            def hp_body(hp, accs):
                hcol = jnp.full((16,), hp, jnp.int32)
                wv = plsc.bitcast(plsc.load_gather(wrows_v, [brow, hcol]),
                                  jnp.bfloat16)
                return tuple(
                    accs[k] + wv * plsc.bitcast(
                        plsc.load_gather(crows_v, [ridx[k], hcol]),
                        jnp.bfloat16)
                    for k in range(K))

            accs = lax.fori_loop(
                0, HPK, hp_body,
                tuple(jnp.zeros((32,), jnp.bfloat16) for _ in range(K)))
            for k in range(K):
                res_v[k, pl.ds(i * CHUNK + sub * 16, 16)] = _fold(accs[k])
        return _

    lax.fori_loop(0, NCHUNK, chunk_body, 0)
    for k in range(K):
        pltpu.sync_copy(res_v.at[k], out_hbm.at[pl.ds(k * B + wid * BPW, BPW)])


@jax.jit
def _sc_dots(wemb_i, cemb_i, widx, cidx):
    mesh = plsc.VectorSubcoreMesh(core_axis_name="c", subcore_axis_name="s",
                                  num_cores=NC, num_subcores=NS)
    return pl.kernel(
        _sc_dots_body,
        out_type=jax.ShapeDtypeStruct((K * B,), jnp.float32),
        mesh=mesh,
        compiler_params=pltpu.CompilerParams(needs_layout_passes=False,
                                             use_tc_tiling_on_sc=False),
        scratch_types=[
            pltpu.VMEM((CHUNK,), jnp.int32),
            pltpu.VMEM((CROWS,), jnp.int32),
            pltpu.VMEM((CHUNK, HPK), jnp.int32),
            pltpu.VMEM((CROWS, HPK), jnp.int32),
            pltpu.VMEM((K, BPW), jnp.float32),
            pltpu.SemaphoreType.DMA,
            pltpu.SemaphoreType.DMA,
            pltpu.SemaphoreType.DMA,
        ],
    )(wemb_i, cemb_i, widx, cidx)


def _loss_body(res_ref, lab_ref, out_ref):
    x = res_ref[...] * lab_ref[...]
    y = jax.nn.log_sigmoid(x)
    out_ref[0, 0] = -jnp.sum(y) / B


def _loss(res2d, lab2d):
    out = pl.pallas_call(
        _loss_body,
        out_shape=jax.ShapeDtypeStruct((1, 1), jnp.float32),
        in_specs=[pl.BlockSpec(memory_space=pltpu.VMEM),
                  pl.BlockSpec(memory_space=pltpu.VMEM)],
        out_specs=pl.BlockSpec(memory_space=pltpu.SMEM),
    )(res2d, lab2d)
    return out[0, 0]


def _pack(table):
    t16 = table.astype(jnp.bfloat16).reshape(table.shape[0], HPK, 2)
    return lax.bitcast_convert_type(t16, jnp.int32)


def kernel(wrd, cntxt, labels, word_emb, context_emb):
    widx = wrd.reshape(B).astype(jnp.int32)
    cidx = cntxt.T.astype(jnp.int32)
    res = _sc_dots(_pack(word_emb), _pack(context_emb), widx, cidx)
    res2d = res.reshape(K * B // 128, 128)
    lab2d = labels.T.reshape(K * B // 128, 128)
    return _loss(res2d, lab2d)
